# baseline (speedup 1.0000x reference)
import functools

import jax
import jax.numpy as jnp
from jax import lax
from jax.experimental import pallas as pl
from jax.experimental.pallas import tpu as pltpu
from jax.experimental.pallas import tpu_sc as plsc

B = 16384
L_ROWS = 100
D = 1024
N_ENC = 5

NC = 2
NS = 16
NW = NC * NS
CHUNK = 64
B_SC = B


NCOPIES = 32
CSTRIDE = 104


def _merge_body(tids_ref, router_ref, enc_ref, out_ref):
    t = tids_ref[0]
    acc = router_ref[t, 0] * enc_ref[0]
    for k in range(1, N_ENC):
        acc += router_ref[t, k] * enc_ref[k]
    out_ref[0, pl.ds(0, L_ROWS), :] = acc


def _merge(tids, router, enc_tables):
    out = pl.pallas_call(
        _merge_body,
        grid=(NCOPIES,),
        in_specs=[
            pl.BlockSpec(memory_space=pltpu.SMEM),
            pl.BlockSpec(memory_space=pltpu.SMEM),
            pl.BlockSpec((N_ENC, L_ROWS, D), lambda j: (0, 0, 0)),
        ],
        out_specs=pl.BlockSpec((1, CSTRIDE, D), lambda j: (j, 0, 0)),
        out_shape=jax.ShapeDtypeStruct((NCOPIES, CSTRIDE, D), jnp.float32),
    )(tids, router, enc_tables)
    return out.reshape(NCOPIES * CSTRIDE, D)


SPLITS = (120, 120, 120, 120, 32)


@functools.cache
def _make_sc_gather():
    b_per_w = B_SC // NW
    mesh = plsc.VectorSubcoreMesh(
        core_axis_name="c", subcore_axis_name="s", num_cores=NC, num_subcores=NS
    )

    @functools.partial(
        pl.kernel,
        out_type=jax.ShapeDtypeStruct((B_SC, D), jnp.float32),
        mesh=mesh,
        scratch_types=[
            pltpu.VMEM((b_per_w,), jnp.int32),
            pltpu.VMEM((max(SPLITS), D), jnp.float32),
            pltpu.SemaphoreType.DMA,
        ],
    )
    def _sc_gather(idx_hbm, rw_hbm, out_hbm, idx_v, rows_v, sem):
        wid = lax.axis_index("s") * NC + lax.axis_index("c")
        base = wid * b_per_w
        pltpu.sync_copy(idx_hbm.at[pl.ds(base, b_per_w)], idx_v)

        off = 0
        for sz in SPLITS:
            pltpu.async_copy(
                rw_hbm.at[idx_v.at[pl.ds(off, sz)]],
                rows_v.at[pl.ds(0, sz)], sem
            ).wait()
            pltpu.sync_copy(rows_v.at[pl.ds(0, sz)],
                            out_hbm.at[pl.ds(base + off, sz)])
            off += sz

    return _sc_gather


def kernel(prompt_token_ids, tids, router, enc_tables, input_ids):
    del input_ids
    rw = _merge(tids, router, enc_tables)
    idx = prompt_token_ids.astype(jnp.int32)
    b_per_w = B_SC // NW
    copy_of_row = (jnp.arange(B, dtype=jnp.int32) // b_per_w) % NCOPIES
    idx = idx + copy_of_row * CSTRIDE
    return _make_sc_gather()(idx, rw)


# speedup vs baseline: 1.0740x; 1.0740x over previous
import functools

import jax
import jax.numpy as jnp
from jax import lax
from jax.experimental import pallas as pl
from jax.experimental.pallas import tpu as pltpu
from jax.experimental.pallas import tpu_sc as plsc

B = 16384
L_ROWS = 100
D = 1024
N_ENC = 5

NC = 2
NS = 16
NW = NC * NS
CHUNK = 64
B_SC = B


NCOPIES = 16
CSTRIDE = 104


def _merge_body(tids_ref, router_ref, enc_ref, out_ref):
    t = tids_ref[0]
    acc = router_ref[t, 0] * enc_ref[0]
    for k in range(1, N_ENC):
        acc += router_ref[t, k] * enc_ref[k]
    out_ref[0, pl.ds(0, L_ROWS), :] = acc


def _merge(tids, router, enc_tables):
    out = pl.pallas_call(
        _merge_body,
        grid=(NCOPIES,),
        in_specs=[
            pl.BlockSpec(memory_space=pltpu.SMEM),
            pl.BlockSpec(memory_space=pltpu.SMEM),
            pl.BlockSpec((N_ENC, L_ROWS, D), lambda j: (0, 0, 0)),
        ],
        out_specs=pl.BlockSpec((1, CSTRIDE, D), lambda j: (j, 0, 0)),
        out_shape=jax.ShapeDtypeStruct((NCOPIES, CSTRIDE, D), jnp.float32),
    )(tids, router, enc_tables)
    return out.reshape(NCOPIES * CSTRIDE, D)


SPLITS = (120, 120, 120, 120, 32)


@functools.cache
def _make_sc_gather():
    b_per_w = B_SC // NW
    mesh = plsc.VectorSubcoreMesh(
        core_axis_name="c", subcore_axis_name="s", num_cores=NC, num_subcores=NS
    )

    @functools.partial(
        pl.kernel,
        out_type=jax.ShapeDtypeStruct((B_SC, D), jnp.float32),
        mesh=mesh,
        scratch_types=[
            pltpu.VMEM((b_per_w,), jnp.int32),
            pltpu.VMEM((max(SPLITS), D), jnp.float32),
            pltpu.SemaphoreType.DMA,
        ],
    )
    def _sc_gather(idx_hbm, rw_hbm, out_hbm, idx_v, rows_v, sem):
        wid = lax.axis_index("s") * NC + lax.axis_index("c")
        base = wid * b_per_w
        pltpu.sync_copy(idx_hbm.at[pl.ds(base, b_per_w)], idx_v)

        off = 0
        for sz in SPLITS:
            pltpu.async_copy(
                rw_hbm.at[idx_v.at[pl.ds(off, sz)]],
                rows_v.at[pl.ds(0, sz)], sem
            ).wait()
            pltpu.sync_copy(rows_v.at[pl.ds(0, sz)],
                            out_hbm.at[pl.ds(base + off, sz)])
            off += sz

    return _sc_gather


def kernel(prompt_token_ids, tids, router, enc_tables, input_ids):
    del input_ids
    rw = _merge(tids, router, enc_tables)
    idx = prompt_token_ids.astype(jnp.int32)
    b_per_w = B_SC // NW
    copy_of_row = (jnp.arange(B, dtype=jnp.int32) // b_per_w) % NCOPIES
    idx = idx + copy_of_row * CSTRIDE
    return _make_sc_gather()(idx, rw)


# speedup vs baseline: 1.0782x; 1.0039x over previous
import functools

import jax
import jax.numpy as jnp
from jax import lax
from jax.experimental import pallas as pl
from jax.experimental.pallas import tpu as pltpu
from jax.experimental.pallas import tpu_sc as plsc

B = 16384
L_ROWS = 100
D = 1024
N_ENC = 5

NC = 2
NS = 16
NW = NC * NS
CHUNK = 64
B_SC = B


NCOPIES = 16
CSTRIDE = 104


def _merge_body(tids_ref, router_ref, enc_ref, out_ref):
    t = tids_ref[0]
    acc = router_ref[t, 0] * enc_ref[0]
    for k in range(1, N_ENC):
        acc += router_ref[t, k] * enc_ref[k]
    out_ref[0, pl.ds(0, L_ROWS), :] = acc


def _merge(tids, router, enc_tables):
    out = pl.pallas_call(
        _merge_body,
        grid=(NCOPIES,),
        in_specs=[
            pl.BlockSpec(memory_space=pltpu.SMEM),
            pl.BlockSpec(memory_space=pltpu.SMEM),
            pl.BlockSpec((N_ENC, L_ROWS, D), lambda j: (0, 0, 0)),
        ],
        out_specs=pl.BlockSpec((1, CSTRIDE, D), lambda j: (j, 0, 0)),
        out_shape=jax.ShapeDtypeStruct((NCOPIES, CSTRIDE, D), jnp.float32),
    )(tids, router, enc_tables)
    return out.reshape(NCOPIES * CSTRIDE, D)


SPLITS = (104, 104, 104, 104, 96)


@functools.cache
def _make_sc_gather():
    b_per_w = B_SC // NW
    mesh = plsc.VectorSubcoreMesh(
        core_axis_name="c", subcore_axis_name="s", num_cores=NC, num_subcores=NS
    )

    @functools.partial(
        pl.kernel,
        out_type=jax.ShapeDtypeStruct((B_SC, D), jnp.float32),
        mesh=mesh,
        scratch_types=[
            pltpu.VMEM((b_per_w,), jnp.int32),
            pltpu.VMEM((max(SPLITS), D), jnp.float32),
            pltpu.SemaphoreType.DMA,
        ],
    )
    def _sc_gather(idx_hbm, rw_hbm, out_hbm, idx_v, rows_v, sem):
        wid = lax.axis_index("s") * NC + lax.axis_index("c")
        base = wid * b_per_w
        pltpu.sync_copy(idx_hbm.at[pl.ds(base, b_per_w)], idx_v)

        off = 0
        for sz in SPLITS:
            pltpu.async_copy(
                rw_hbm.at[idx_v.at[pl.ds(off, sz)]],
                rows_v.at[pl.ds(0, sz)], sem
            ).wait()
            pltpu.sync_copy(rows_v.at[pl.ds(0, sz)],
                            out_hbm.at[pl.ds(base + off, sz)])
            off += sz

    return _sc_gather


def kernel(prompt_token_ids, tids, router, enc_tables, input_ids):
    del input_ids
    rw = _merge(tids, router, enc_tables)
    idx = prompt_token_ids.astype(jnp.int32)
    b_per_w = B_SC // NW
    copy_of_row = (jnp.arange(B, dtype=jnp.int32) // b_per_w) % NCOPIES
    idx = idx + copy_of_row * CSTRIDE
    return _make_sc_gather()(idx, rw)


# speedup vs baseline: 1.0823x; 1.0038x over previous
import functools

import jax
import jax.numpy as jnp
from jax import lax
from jax.experimental import pallas as pl
from jax.experimental.pallas import tpu as pltpu
from jax.experimental.pallas import tpu_sc as plsc

B = 16384
L_ROWS = 100
D = 1024
N_ENC = 5

NC = 2
NS = 16
NW = NC * NS
NCOPIES = 16
CSTRIDE = 104


def _merge_body(tids_ref, router_ref, enc_ref, out_ref):
    t = tids_ref[0]
    acc = router_ref[t, 0] * enc_ref[0]
    for k in range(1, N_ENC):
        acc += router_ref[t, k] * enc_ref[k]
    out_ref[0, pl.ds(0, L_ROWS), :] = acc


def _merge(tids, router, enc_tables):
    out = pl.pallas_call(
        _merge_body,
        grid=(NCOPIES,),
        in_specs=[
            pl.BlockSpec(memory_space=pltpu.SMEM),
            pl.BlockSpec(memory_space=pltpu.SMEM),
            pl.BlockSpec((N_ENC, L_ROWS, D), lambda j: (0, 0, 0)),
        ],
        out_specs=pl.BlockSpec((1, CSTRIDE, D), lambda j: (j, 0, 0)),
        out_shape=jax.ShapeDtypeStruct((NCOPIES, CSTRIDE, D), jnp.float32),
    )(tids, router, enc_tables)
    return out.reshape(NCOPIES * CSTRIDE, D)


SPLITS = (104, 104, 104, 104, 96)


@functools.cache
def _make_sc_gather():
    b_per_w = B // NW
    mesh = plsc.VectorSubcoreMesh(
        core_axis_name="c", subcore_axis_name="s", num_cores=NC, num_subcores=NS
    )

    @functools.partial(
        pl.kernel,
        out_type=jax.ShapeDtypeStruct((B, D), jnp.float32),
        mesh=mesh,
        scratch_types=[
            pltpu.VMEM((b_per_w,), jnp.int32),
            pltpu.VMEM((max(SPLITS), D), jnp.float32),
            pltpu.SemaphoreType.DMA,
        ],
    )
    def _sc_gather(idx_hbm, rw_hbm, out_hbm, idx_v, rows_v, sem):
        wid = lax.axis_index("s") * NC + lax.axis_index("c")
        base = wid * b_per_w
        pltpu.sync_copy(idx_hbm.at[pl.ds(base, b_per_w)], idx_v)

        off = 0
        for sz in SPLITS:
            pltpu.async_copy(
                rw_hbm.at[idx_v.at[pl.ds(off, sz)]],
                rows_v.at[pl.ds(0, sz)], sem
            ).wait()
            pltpu.sync_copy(rows_v.at[pl.ds(0, sz)],
                            out_hbm.at[pl.ds(base + off, sz)])
            off += sz

    return _sc_gather


def kernel(prompt_token_ids, tids, router, enc_tables, input_ids):
    del input_ids
    rw = _merge(tids, router, enc_tables)
    idx = prompt_token_ids.astype(jnp.int32)
    b_per_w = B // NW
    copy_of_row = (jnp.arange(B, dtype=jnp.int32) // b_per_w) % NCOPIES
    idx = idx + copy_of_row * CSTRIDE
    return _make_sc_gather()(idx, rw)
